# baseline (speedup 1.0000x reference)
import functools
import jax
import jax.numpy as jnp
from jax import lax
from jax.experimental import pallas as pl
from jax.experimental.pallas import tpu as pltpu
from jax.experimental.pallas import tpu_sc as plsc

N = 10000
E = 320000
D = 128

NC = 2
NS = 16
NW = NC * NS
CHUNK = 128
G = 8
NCHUNKS = 2560
EPAD = NCHUNKS * CHUNK
NB = NCHUNKS // G
BITERS = NB // NW
ZCH = 80
NZ = N // ZCH
ZITERS = (NZ + NS - 1) // NS

ROW_BLOCK = 1000


def _spmm_body(h_hbm, pack_hbm, z_hbm, out_hbm,
               idxb, src_v, dst_v, rows_v, acc_shared, sem):
    cid = lax.axis_index("c")
    sid = lax.axis_index("s")
    wid = sid * NC + cid

    for i in range(ZITERS):
        c = sid + i * NS

        @pl.when(c < NZ)
        def _():
            pltpu.sync_copy(z_hbm, acc_shared.at[pl.ds(c * ZCH, ZCH)])

    plsc.subcore_barrier()

    def block_iter(i, carry):
        b = wid + i * NW
        pltpu.sync_copy(pack_hbm.at[pl.ds(b * G * 3, G * 3)], idxb)

        def chunk_body(g, cc):
            for j in range(CHUNK // 16):
                sl = pl.ds(j * 16, 16)
                src_v[sl] = idxb[3 * g + 0, sl]
                dst_v[sl] = idxb[3 * g + 1, sl]
            pltpu.async_copy(h_hbm.at[src_v], rows_v, sem).wait()

            def scale(e, inner):
                wbits = plsc.load_gather(
                    idxb, [jnp.full((16,), 3 * g + 2, jnp.int32),
                           jnp.full((16,), e, jnp.int32)])
                wvec = plsc.bitcast(wbits, jnp.float32)
                for j in range(D // 16):
                    sl = pl.ds(j * 16, 16)
                    rows_v[e, sl] = rows_v[e, sl] * wvec
                return inner

            lax.fori_loop(0, CHUNK, scale, 0, unroll=2)
            pltpu.sync_copy(rows_v, acc_shared.at[dst_v], add=True)
            return cc

        lax.fori_loop(0, G, chunk_body, 0)
        return carry

    lax.fori_loop(0, BITERS, block_iter, 0)
    plsc.subcore_barrier()

    for i in range(ZITERS):
        c = sid + i * NS

        @pl.when(c < NZ)
        def _():
            pltpu.sync_copy(acc_shared.at[pl.ds(c * ZCH, ZCH)],
                            out_hbm.at[cid, pl.ds(c * ZCH, ZCH)])


@jax.jit
def _spmm(h, pack, zeros):
    mesh = plsc.VectorSubcoreMesh(core_axis_name="c", subcore_axis_name="s")
    f = pl.kernel(
        _spmm_body,
        out_type=jax.ShapeDtypeStruct((NC, N, D), jnp.float32),
        mesh=mesh,
        compiler_params=pltpu.CompilerParams(needs_layout_passes=False),
        scratch_types=[
            pltpu.VMEM((G * 3, CHUNK), jnp.int32),
            pltpu.VMEM((CHUNK,), jnp.int32),
            pltpu.VMEM((CHUNK,), jnp.int32),
            pltpu.VMEM((CHUNK, D), jnp.float32),
            pltpu.VMEM_SHARED((N, D), jnp.float32),
            pltpu.SemaphoreType.DMA,
        ],
    )
    return f(h, pack, zeros)


def _fc1_body(x_ref, w_ref, b_ref, o_ref):
    o_ref[...] = (
        jnp.dot(x_ref[...], w_ref[...], preferred_element_type=jnp.float32)
        + b_ref[...]
    )


@jax.jit
def _fc1(x, w, b):
    return pl.pallas_call(
        _fc1_body,
        grid=(N // ROW_BLOCK,),
        in_specs=[
            pl.BlockSpec((ROW_BLOCK, D), lambda i: (i, 0)),
            pl.BlockSpec((D, D), lambda i: (0, 0)),
            pl.BlockSpec((1, D), lambda i: (0, 0)),
        ],
        out_specs=pl.BlockSpec((ROW_BLOCK, D), lambda i: (i, 0)),
        out_shape=jax.ShapeDtypeStruct((N, D), jnp.float32),
    )(x, w, b)


def _fc2_body(p_ref, w_ref, b_ref, o_ref):
    h = jnp.maximum(p_ref[0] + p_ref[1], 0.0)
    y = jnp.dot(h, w_ref[...], preferred_element_type=jnp.float32) + b_ref[...]
    m = jnp.max(y, axis=1, keepdims=True)
    s = y - m
    o_ref[...] = s - jnp.log(jnp.sum(jnp.exp(s), axis=1, keepdims=True))


@jax.jit
def _fc2(parts, w, b):
    return pl.pallas_call(
        _fc2_body,
        grid=(N // ROW_BLOCK,),
        in_specs=[
            pl.BlockSpec((NC, ROW_BLOCK, D), lambda i: (0, i, 0)),
            pl.BlockSpec((D, D), lambda i: (0, 0)),
            pl.BlockSpec((1, D), lambda i: (0, 0)),
        ],
        out_specs=pl.BlockSpec((ROW_BLOCK, D), lambda i: (i, 0)),
        out_shape=jax.ShapeDtypeStruct((N, D), jnp.float32),
    )(parts, w, b)


def kernel(features, edge_index, edge_weight, W1, b1, W2, b2):
    pad = EPAD - E
    src = jnp.pad(edge_index[0].astype(jnp.int32), (0, pad))
    dst = jnp.pad(edge_index[1].astype(jnp.int32), (0, pad))
    wbits = jax.lax.bitcast_convert_type(
        jnp.pad(edge_weight, (0, pad)), jnp.int32)
    pack = jnp.stack([src.reshape(NCHUNKS, CHUNK),
                      dst.reshape(NCHUNKS, CHUNK),
                      wbits.reshape(NCHUNKS, CHUNK)],
                     axis=1).reshape(NCHUNKS * 3, CHUNK)
    h = _fc1(features, W1, b1.reshape(1, D))
    zeros = jnp.zeros((ZCH, D), jnp.float32)
    parts = _spmm(h, pack, zeros)
    return _fc2(parts, W2, b2.reshape(1, D))


# speedup vs baseline: 1.0005x; 1.0005x over previous
import functools
import jax
import jax.numpy as jnp
from jax import lax
from jax.experimental import pallas as pl
from jax.experimental.pallas import tpu as pltpu
from jax.experimental.pallas import tpu_sc as plsc

N = 10000
E = 320000
D = 128

NC = 2
NS = 16
NW = NC * NS
CHUNK = 128
G = 8
NCHUNKS = 2560
EPAD = NCHUNKS * CHUNK
NB = NCHUNKS // G
BITERS = NB // NW
ZCH = 80
NZ = N // ZCH
ZITERS = (NZ + NS - 1) // NS

ROW_BLOCK = 1000


def _spmm_body(h_hbm, pack_hbm, z_hbm, out_hbm,
               idxb, src_v, dst_v, rows_v, acc_shared, sem):
    BW = 3 * CHUNK
    cid = lax.axis_index("c")
    sid = lax.axis_index("s")
    wid = sid * NC + cid

    for i in range(ZITERS):
        c = sid + i * NS

        @pl.when(c < NZ)
        def _():
            pltpu.sync_copy(z_hbm, acc_shared.at[pl.ds(c * ZCH, ZCH)])

    plsc.subcore_barrier()

    def block_iter(i, carry):
        b = wid + i * NW
        pltpu.sync_copy(pack_hbm.at[pl.ds(b * (G * BW), G * BW)], idxb)

        def chunk_body(g, cc):
            goff = g * BW
            for j in range(CHUNK // 16):
                sl = pl.ds(j * 16, 16)
                src_v[sl] = idxb[pl.ds(goff + j * 16, 16)]
                dst_v[sl] = idxb[pl.ds(goff + CHUNK + j * 16, 16)]
            pltpu.async_copy(h_hbm.at[src_v], rows_v, sem).wait()

            def scale(e, inner):
                wbits = plsc.load_gather(
                    idxb, [jnp.full((16,), goff + 2 * CHUNK, jnp.int32) + e])
                wvec = plsc.bitcast(wbits, jnp.float32)
                for j in range(D // 16):
                    sl = pl.ds(j * 16, 16)
                    rows_v[e, sl] = rows_v[e, sl] * wvec
                return inner

            lax.fori_loop(0, CHUNK, scale, 0, unroll=2)
            pltpu.sync_copy(rows_v, acc_shared.at[dst_v], add=True)
            return cc

        lax.fori_loop(0, G, chunk_body, 0)
        return carry

    lax.fori_loop(0, BITERS, block_iter, 0)
    plsc.subcore_barrier()

    for i in range(ZITERS):
        c = sid + i * NS

        @pl.when(c < NZ)
        def _():
            pltpu.sync_copy(acc_shared.at[pl.ds(c * ZCH, ZCH)],
                            out_hbm.at[cid, pl.ds(c * ZCH, ZCH)])


@jax.jit
def _spmm(h, pack, zeros):
    mesh = plsc.VectorSubcoreMesh(core_axis_name="c", subcore_axis_name="s")
    f = pl.kernel(
        _spmm_body,
        out_type=jax.ShapeDtypeStruct((NC, N, D), jnp.float32),
        mesh=mesh,
        compiler_params=pltpu.CompilerParams(needs_layout_passes=False),
        scratch_types=[
            pltpu.VMEM((G * 3 * CHUNK,), jnp.int32),
            pltpu.VMEM((CHUNK,), jnp.int32),
            pltpu.VMEM((CHUNK,), jnp.int32),
            pltpu.VMEM((CHUNK, D), jnp.float32),
            pltpu.VMEM_SHARED((N, D), jnp.float32),
            pltpu.SemaphoreType.DMA,
        ],
    )
    return f(h, pack, zeros)


def _fc1_body(x_ref, w_ref, b_ref, o_ref):
    o_ref[...] = (
        jnp.dot(x_ref[...], w_ref[...], preferred_element_type=jnp.float32)
        + b_ref[...]
    )


@jax.jit
def _fc1(x, w, b):
    return pl.pallas_call(
        _fc1_body,
        grid=(N // ROW_BLOCK,),
        in_specs=[
            pl.BlockSpec((ROW_BLOCK, D), lambda i: (i, 0)),
            pl.BlockSpec((D, D), lambda i: (0, 0)),
            pl.BlockSpec((1, D), lambda i: (0, 0)),
        ],
        out_specs=pl.BlockSpec((ROW_BLOCK, D), lambda i: (i, 0)),
        out_shape=jax.ShapeDtypeStruct((N, D), jnp.float32),
    )(x, w, b)


def _fc2_body(p_ref, w_ref, b_ref, o_ref):
    h = jnp.maximum(p_ref[0] + p_ref[1], 0.0)
    y = jnp.dot(h, w_ref[...], preferred_element_type=jnp.float32) + b_ref[...]
    m = jnp.max(y, axis=1, keepdims=True)
    s = y - m
    o_ref[...] = s - jnp.log(jnp.sum(jnp.exp(s), axis=1, keepdims=True))


@jax.jit
def _fc2(parts, w, b):
    return pl.pallas_call(
        _fc2_body,
        grid=(N // ROW_BLOCK,),
        in_specs=[
            pl.BlockSpec((NC, ROW_BLOCK, D), lambda i: (0, i, 0)),
            pl.BlockSpec((D, D), lambda i: (0, 0)),
            pl.BlockSpec((1, D), lambda i: (0, 0)),
        ],
        out_specs=pl.BlockSpec((ROW_BLOCK, D), lambda i: (i, 0)),
        out_shape=jax.ShapeDtypeStruct((N, D), jnp.float32),
    )(parts, w, b)


def kernel(features, edge_index, edge_weight, W1, b1, W2, b2):
    pad = EPAD - E
    src = jnp.pad(edge_index[0].astype(jnp.int32), (0, pad))
    dst = jnp.pad(edge_index[1].astype(jnp.int32), (0, pad))
    wbits = jax.lax.bitcast_convert_type(
        jnp.pad(edge_weight, (0, pad)), jnp.int32)
    pack = jnp.stack([src.reshape(NCHUNKS, CHUNK),
                      dst.reshape(NCHUNKS, CHUNK),
                      wbits.reshape(NCHUNKS, CHUNK)],
                     axis=1).reshape(NCHUNKS * 3 * CHUNK)
    h = _fc1(features, W1, b1.reshape(1, D))
    zeros = jnp.zeros((ZCH, D), jnp.float32)
    parts = _spmm(h, pack, zeros)
    return _fc2(parts, W2, b2.reshape(1, D))


# speedup vs baseline: 1.7517x; 1.7508x over previous
import functools
import jax
import jax.numpy as jnp
from jax import lax
from jax.experimental import pallas as pl
from jax.experimental.pallas import tpu as pltpu
from jax.experimental.pallas import tpu_sc as plsc

N = 10000
E = 320000
D = 128

NC = 2
NS = 16
NW = NC * NS
CHUNK = 128
G = 8
NCHUNKS = 2560
EPAD = NCHUNKS * CHUNK
NB = NCHUNKS // G
BITERS = NB // NW
ZCH = 80
NZ = N // ZCH
ZITERS = (NZ + NS - 1) // NS

ROW_BLOCK = 1000


def _spmm_body(h_hbm, pack_hbm, z_hbm, out_hbm,
               idxb, src_v, dst_v, rows_v, acc_shared, sem):
    BW = 3 * CHUNK
    cid = lax.axis_index("c")
    sid = lax.axis_index("s")
    wid = sid * NC + cid

    for i in range(ZITERS):
        c = sid + i * NS

        @pl.when(c < NZ)
        def _():
            pltpu.sync_copy(z_hbm, acc_shared.at[pl.ds(c * ZCH, ZCH)])

    plsc.subcore_barrier()

    def block_iter(i, carry):
        b = wid + i * NW
        pltpu.sync_copy(pack_hbm.at[pl.ds(b * (G * BW), G * BW)], idxb)

        def chunk_body(g, cc):
            goff = g * BW
            for j in range(CHUNK // 16):
                sl = pl.ds(j * 16, 16)
                src_v[sl] = idxb[pl.ds(goff + j * 16, 16)]
                dst_v[sl] = idxb[pl.ds(goff + CHUNK + j * 16, 16)]
            pltpu.async_copy(h_hbm.at[src_v], rows_v, sem).wait()

            def scale(e, inner):
                wbits = plsc.load_gather(
                    idxb, [jnp.full((16,), goff + 2 * CHUNK, jnp.int32) + e])
                wvec = plsc.bitcast(wbits, jnp.float32)
                for j in range(D // 16):
                    sl = pl.ds(j * 16, 16)
                    rows_v[e, sl] = rows_v[e, sl] * wvec
                return inner

            lax.fori_loop(0, CHUNK, scale, 0, unroll=2)
            pltpu.sync_copy(rows_v, acc_shared.at[dst_v], add=True)
            return cc

        lax.fori_loop(0, G, chunk_body, 0)
        return carry

    lax.fori_loop(0, BITERS, block_iter, 0)
    plsc.subcore_barrier()

    for i in range(ZITERS):
        c = sid + i * NS

        @pl.when(c < NZ)
        def _():
            pltpu.sync_copy(acc_shared.at[pl.ds(c * ZCH, ZCH)],
                            out_hbm.at[cid, pl.ds(c * ZCH, ZCH)])


@jax.jit
def _spmm(h, pack, zeros):
    mesh = plsc.VectorSubcoreMesh(core_axis_name="c", subcore_axis_name="s")
    f = pl.kernel(
        _spmm_body,
        out_type=jax.ShapeDtypeStruct((NC, N, D), jnp.float32),
        mesh=mesh,
        compiler_params=pltpu.CompilerParams(needs_layout_passes=False),
        scratch_types=[
            pltpu.VMEM((G * 3 * CHUNK,), jnp.int32),
            pltpu.VMEM((CHUNK,), jnp.int32),
            pltpu.VMEM((CHUNK,), jnp.int32),
            pltpu.VMEM((CHUNK, D), jnp.float32),
            pltpu.VMEM_SHARED((N, D), jnp.float32),
            pltpu.SemaphoreType.DMA,
        ],
    )
    return f(h, pack, zeros)


def _fc1_body(x_ref, w_ref, b_ref, o_ref):
    o_ref[...] = (
        jnp.dot(x_ref[...], w_ref[...], preferred_element_type=jnp.float32)
        + b_ref[...]
    )


@jax.jit
def _fc1(x, w, b):
    return pl.pallas_call(
        _fc1_body,
        grid=(N // ROW_BLOCK,),
        in_specs=[
            pl.BlockSpec((ROW_BLOCK, D), lambda i: (i, 0)),
            pl.BlockSpec((D, D), lambda i: (0, 0)),
            pl.BlockSpec((1, D), lambda i: (0, 0)),
        ],
        out_specs=pl.BlockSpec((ROW_BLOCK, D), lambda i: (i, 0)),
        out_shape=jax.ShapeDtypeStruct((N, D), jnp.float32),
    )(x, w, b)


def _fc2_body(p_ref, w_ref, b_ref, o_ref):
    h = jnp.maximum(p_ref[0] + p_ref[1], 0.0)
    y = jnp.dot(h, w_ref[...], preferred_element_type=jnp.float32) + b_ref[...]
    m = jnp.max(y, axis=1, keepdims=True)
    s = y - m
    o_ref[...] = s - jnp.log(jnp.sum(jnp.exp(s), axis=1, keepdims=True))


@jax.jit
def _fc2(parts, w, b):
    return pl.pallas_call(
        _fc2_body,
        grid=(N // ROW_BLOCK,),
        in_specs=[
            pl.BlockSpec((NC, ROW_BLOCK, D), lambda i: (0, i, 0)),
            pl.BlockSpec((D, D), lambda i: (0, 0)),
            pl.BlockSpec((1, D), lambda i: (0, 0)),
        ],
        out_specs=pl.BlockSpec((ROW_BLOCK, D), lambda i: (i, 0)),
        out_shape=jax.ShapeDtypeStruct((N, D), jnp.float32),
    )(parts, w, b)


def kernel(features, edge_index, edge_weight, W1, b1, W2, b2):
    pad = EPAD - E
    fill = (jnp.arange(pad, dtype=jnp.int32) * 16) % N
    src = jnp.concatenate([edge_index[0].astype(jnp.int32), fill])
    dst = jnp.concatenate([edge_index[1].astype(jnp.int32), fill])
    wbits = jax.lax.bitcast_convert_type(
        jnp.pad(edge_weight, (0, pad)), jnp.int32)
    pack = jnp.stack([src.reshape(NCHUNKS, CHUNK),
                      dst.reshape(NCHUNKS, CHUNK),
                      wbits.reshape(NCHUNKS, CHUNK)],
                     axis=1).reshape(NCHUNKS * 3 * CHUNK)
    h = _fc1(features, W1, b1.reshape(1, D))
    zeros = jnp.zeros((ZCH, D), jnp.float32)
    parts = _spmm(h, pack, zeros)
    return _fc2(parts, W2, b2.reshape(1, D))


# speedup vs baseline: 2.3405x; 1.3362x over previous
import functools
import jax
import jax.numpy as jnp
from jax import lax
from jax.experimental import pallas as pl
from jax.experimental.pallas import tpu as pltpu
from jax.experimental.pallas import tpu_sc as plsc

N = 10000
E = 320000
D = 128

NC = 2
NS = 16
NW = NC * NS
CHUNK = 128
G = 8
NCHUNKS = 2560
EPAD = NCHUNKS * CHUNK
NB = NCHUNKS // G
BITERS = NB // NW
ZCH = 80
NZ = N // ZCH
ZITERS = (NZ + NS - 1) // NS

ROW_BLOCK = 1000


def _spmm_body(h_hbm, pack_hbm, z_hbm, out_hbm,
               idxb, src0, src1, dst0, dst1, rows0, rows1, acc_shared,
               sem0, sem1):
    BW = 3 * CHUNK
    srcv = (src0, src1)
    dstv = (dst0, dst1)
    rows = (rows0, rows1)
    sems = (sem0, sem1)
    cid = lax.axis_index("c")
    sid = lax.axis_index("s")
    wid = sid * NC + cid

    for i in range(ZITERS):
        c = sid + i * NS

        @pl.when(c < NZ)
        def _():
            pltpu.sync_copy(z_hbm, acc_shared.at[pl.ds(c * ZCH, ZCH)])

    plsc.subcore_barrier()

    def block_iter(i, carry):
        b = wid + i * NW
        pltpu.sync_copy(pack_hbm.at[pl.ds(b * (G * BW), G * BW)], idxb)

        def stage(g):
            k = g % 2
            goff = g * BW
            for j in range(CHUNK // 16):
                sl = pl.ds(j * 16, 16)
                srcv[k][sl] = idxb[pl.ds(goff + j * 16, 16)]
                dstv[k][sl] = idxb[pl.ds(goff + CHUNK + j * 16, 16)]
            return pltpu.async_copy(h_hbm.at[srcv[k]], rows[k], sems[k])

        gd = [None] * G
        gd[0] = stage(0)
        for g in range(G):
            k = g % 2
            if g + 1 < G:
                gd[g + 1] = stage(g + 1)
            gd[g].wait()
            rows_v = rows[k]
            goff = g * BW

            def scale(e, inner):
                wbits = plsc.load_gather(
                    idxb, [jnp.full((16,), goff + 2 * CHUNK, jnp.int32) + e])
                wvec = plsc.bitcast(wbits, jnp.float32)
                for j in range(D // 16):
                    sl = pl.ds(j * 16, 16)
                    rows_v[e, sl] = rows_v[e, sl] * wvec
                return inner

            lax.fori_loop(0, CHUNK, scale, 0, unroll=2)
            pltpu.sync_copy(rows_v, acc_shared.at[dstv[k]], add=True)
        return carry

    lax.fori_loop(0, BITERS, block_iter, 0)
    plsc.subcore_barrier()

    for i in range(ZITERS):
        c = sid + i * NS

        @pl.when(c < NZ)
        def _():
            pltpu.sync_copy(acc_shared.at[pl.ds(c * ZCH, ZCH)],
                            out_hbm.at[cid, pl.ds(c * ZCH, ZCH)])


@jax.jit
def _spmm(h, pack, zeros):
    mesh = plsc.VectorSubcoreMesh(core_axis_name="c", subcore_axis_name="s")
    f = pl.kernel(
        _spmm_body,
        out_type=jax.ShapeDtypeStruct((NC, N, D), jnp.float32),
        mesh=mesh,
        compiler_params=pltpu.CompilerParams(needs_layout_passes=False),
        scratch_types=[
            pltpu.VMEM((G * 3 * CHUNK,), jnp.int32),
            pltpu.VMEM((CHUNK,), jnp.int32),
            pltpu.VMEM((CHUNK,), jnp.int32),
            pltpu.VMEM((CHUNK,), jnp.int32),
            pltpu.VMEM((CHUNK,), jnp.int32),
            pltpu.VMEM((CHUNK, D), jnp.float32),
            pltpu.VMEM((CHUNK, D), jnp.float32),
            pltpu.VMEM_SHARED((N, D), jnp.float32),
            pltpu.SemaphoreType.DMA,
            pltpu.SemaphoreType.DMA,
        ],
    )
    return f(h, pack, zeros)


def _fc1_body(x_ref, w_ref, b_ref, o_ref):
    o_ref[...] = (
        jnp.dot(x_ref[...], w_ref[...], preferred_element_type=jnp.float32)
        + b_ref[...]
    )


@jax.jit
def _fc1(x, w, b):
    return pl.pallas_call(
        _fc1_body,
        grid=(N // ROW_BLOCK,),
        in_specs=[
            pl.BlockSpec((ROW_BLOCK, D), lambda i: (i, 0)),
            pl.BlockSpec((D, D), lambda i: (0, 0)),
            pl.BlockSpec((1, D), lambda i: (0, 0)),
        ],
        out_specs=pl.BlockSpec((ROW_BLOCK, D), lambda i: (i, 0)),
        out_shape=jax.ShapeDtypeStruct((N, D), jnp.float32),
    )(x, w, b)


def _fc2_body(p_ref, w_ref, b_ref, o_ref):
    h = jnp.maximum(p_ref[0] + p_ref[1], 0.0)
    y = jnp.dot(h, w_ref[...], preferred_element_type=jnp.float32) + b_ref[...]
    m = jnp.max(y, axis=1, keepdims=True)
    s = y - m
    o_ref[...] = s - jnp.log(jnp.sum(jnp.exp(s), axis=1, keepdims=True))


@jax.jit
def _fc2(parts, w, b):
    return pl.pallas_call(
        _fc2_body,
        grid=(N // ROW_BLOCK,),
        in_specs=[
            pl.BlockSpec((NC, ROW_BLOCK, D), lambda i: (0, i, 0)),
            pl.BlockSpec((D, D), lambda i: (0, 0)),
            pl.BlockSpec((1, D), lambda i: (0, 0)),
        ],
        out_specs=pl.BlockSpec((ROW_BLOCK, D), lambda i: (i, 0)),
        out_shape=jax.ShapeDtypeStruct((N, D), jnp.float32),
    )(parts, w, b)


def kernel(features, edge_index, edge_weight, W1, b1, W2, b2):
    pad = EPAD - E
    fill = (jnp.arange(pad, dtype=jnp.int32) * 16) % N
    src = jnp.concatenate([edge_index[0].astype(jnp.int32), fill])
    dst = jnp.concatenate([edge_index[1].astype(jnp.int32), fill])
    wbits = jax.lax.bitcast_convert_type(
        jnp.pad(edge_weight, (0, pad)), jnp.int32)
    pack = jnp.stack([src.reshape(NCHUNKS, CHUNK),
                      dst.reshape(NCHUNKS, CHUNK),
                      wbits.reshape(NCHUNKS, CHUNK)],
                     axis=1).reshape(NCHUNKS * 3 * CHUNK)
    h = _fc1(features, W1, b1.reshape(1, D))
    zeros = jnp.zeros((ZCH, D), jnp.float32)
    parts = _spmm(h, pack, zeros)
    return _fc2(parts, W2, b2.reshape(1, D))


# speedup vs baseline: 2.3487x; 1.0035x over previous
import functools
import jax
import jax.numpy as jnp
from jax import lax
from jax.experimental import pallas as pl
from jax.experimental.pallas import tpu as pltpu
from jax.experimental.pallas import tpu_sc as plsc

N = 10000
E = 320000
D = 128

NC = 2
NS = 16
NW = NC * NS
CHUNK = 128
G = 8
NCHUNKS = 2560
EPAD = NCHUNKS * CHUNK
NB = NCHUNKS // G
BITERS = NB // NW
ZCH = 80
NZ = N // ZCH
ZITERS = (NZ + NS - 1) // NS

ROW_BLOCK = 1000


def _spmm_body(h_hbm, pack_hbm, z_hbm, out_hbm,
               idxb, src0, src1, dst0, dst1, rows0, rows1, acc_shared,
               sem0, sem1):
    BW = 3 * CHUNK
    srcv = (src0, src1)
    dstv = (dst0, dst1)
    rows = (rows0, rows1)
    sems = (sem0, sem1)
    cid = lax.axis_index("c")
    sid = lax.axis_index("s")
    wid = sid * NC + cid

    for i in range(ZITERS):
        c = sid + i * NS

        @pl.when(c < NZ)
        def _():
            pltpu.sync_copy(z_hbm, acc_shared.at[pl.ds(c * ZCH, ZCH)])

    plsc.subcore_barrier()

    def block_iter(i, carry):
        b = wid + i * NW
        pltpu.sync_copy(pack_hbm.at[pl.ds(b * (G * BW), G * BW)], idxb)

        def stage(g):
            k = g % 2
            goff = g * BW
            for j in range(CHUNK // 16):
                sl = pl.ds(j * 16, 16)
                srcv[k][sl] = idxb[pl.ds(goff + j * 16, 16)]
                dstv[k][sl] = idxb[pl.ds(goff + CHUNK + j * 16, 16)]
            return pltpu.async_copy(h_hbm.at[srcv[k]], rows[k], sems[k])

        gd = [None] * G
        gd[0] = stage(0)
        for g in range(G):
            k = g % 2
            if g + 1 < G:
                gd[g + 1] = stage(g + 1)
            gd[g].wait()
            rows_v = rows[k]
            goff = g * BW

            def scale(e, inner):
                wbits = plsc.load_gather(
                    idxb, [jnp.full((16,), goff + 2 * CHUNK, jnp.int32) + e])
                wvec = plsc.bitcast(wbits, jnp.float32)
                for j in range(D // 16):
                    sl = pl.ds(j * 16, 16)
                    rows_v[e, sl] = rows_v[e, sl] * wvec
                return inner

            lax.fori_loop(0, CHUNK, scale, 0, unroll=4)
            pltpu.sync_copy(rows_v, acc_shared.at[dstv[k]], add=True)
        return carry

    lax.fori_loop(0, BITERS, block_iter, 0)
    plsc.subcore_barrier()

    for i in range(ZITERS):
        c = sid + i * NS

        @pl.when(c < NZ)
        def _():
            pltpu.sync_copy(acc_shared.at[pl.ds(c * ZCH, ZCH)],
                            out_hbm.at[cid, pl.ds(c * ZCH, ZCH)])


@jax.jit
def _spmm(h, pack, zeros):
    mesh = plsc.VectorSubcoreMesh(core_axis_name="c", subcore_axis_name="s")
    f = pl.kernel(
        _spmm_body,
        out_type=jax.ShapeDtypeStruct((NC, N, D), jnp.float32),
        mesh=mesh,
        compiler_params=pltpu.CompilerParams(needs_layout_passes=False),
        scratch_types=[
            pltpu.VMEM((G * 3 * CHUNK,), jnp.int32),
            pltpu.VMEM((CHUNK,), jnp.int32),
            pltpu.VMEM((CHUNK,), jnp.int32),
            pltpu.VMEM((CHUNK,), jnp.int32),
            pltpu.VMEM((CHUNK,), jnp.int32),
            pltpu.VMEM((CHUNK, D), jnp.float32),
            pltpu.VMEM((CHUNK, D), jnp.float32),
            pltpu.VMEM_SHARED((N, D), jnp.float32),
            pltpu.SemaphoreType.DMA,
            pltpu.SemaphoreType.DMA,
        ],
    )
    return f(h, pack, zeros)


def _fc1_body(x_ref, w_ref, b_ref, o_ref):
    o_ref[...] = (
        jnp.dot(x_ref[...], w_ref[...], preferred_element_type=jnp.float32)
        + b_ref[...]
    )


@jax.jit
def _fc1(x, w, b):
    return pl.pallas_call(
        _fc1_body,
        grid=(N // ROW_BLOCK,),
        in_specs=[
            pl.BlockSpec((ROW_BLOCK, D), lambda i: (i, 0)),
            pl.BlockSpec((D, D), lambda i: (0, 0)),
            pl.BlockSpec((1, D), lambda i: (0, 0)),
        ],
        out_specs=pl.BlockSpec((ROW_BLOCK, D), lambda i: (i, 0)),
        out_shape=jax.ShapeDtypeStruct((N, D), jnp.float32),
    )(x, w, b)


def _fc2_body(p_ref, w_ref, b_ref, o_ref):
    h = jnp.maximum(p_ref[0] + p_ref[1], 0.0)
    y = jnp.dot(h, w_ref[...], preferred_element_type=jnp.float32) + b_ref[...]
    m = jnp.max(y, axis=1, keepdims=True)
    s = y - m
    o_ref[...] = s - jnp.log(jnp.sum(jnp.exp(s), axis=1, keepdims=True))


@jax.jit
def _fc2(parts, w, b):
    return pl.pallas_call(
        _fc2_body,
        grid=(N // ROW_BLOCK,),
        in_specs=[
            pl.BlockSpec((NC, ROW_BLOCK, D), lambda i: (0, i, 0)),
            pl.BlockSpec((D, D), lambda i: (0, 0)),
            pl.BlockSpec((1, D), lambda i: (0, 0)),
        ],
        out_specs=pl.BlockSpec((ROW_BLOCK, D), lambda i: (i, 0)),
        out_shape=jax.ShapeDtypeStruct((N, D), jnp.float32),
    )(parts, w, b)


def kernel(features, edge_index, edge_weight, W1, b1, W2, b2):
    pad = EPAD - E
    fill = (jnp.arange(pad, dtype=jnp.int32) * 16) % N
    src = jnp.concatenate([edge_index[0].astype(jnp.int32), fill])
    dst = jnp.concatenate([edge_index[1].astype(jnp.int32), fill])
    wbits = jax.lax.bitcast_convert_type(
        jnp.pad(edge_weight, (0, pad)), jnp.int32)
    pack = jnp.stack([src.reshape(NCHUNKS, CHUNK),
                      dst.reshape(NCHUNKS, CHUNK),
                      wbits.reshape(NCHUNKS, CHUNK)],
                     axis=1).reshape(NCHUNKS * 3 * CHUNK)
    h = _fc1(features, W1, b1.reshape(1, D))
    zeros = jnp.zeros((ZCH, D), jnp.float32)
    parts = _spmm(h, pack, zeros)
    return _fc2(parts, W2, b2.reshape(1, D))


# speedup vs baseline: 2.4500x; 1.0431x over previous
import functools
import jax
import jax.numpy as jnp
from jax import lax
from jax.experimental import pallas as pl
from jax.experimental.pallas import tpu as pltpu
from jax.experimental.pallas import tpu_sc as plsc

N = 10000
E = 320000
D = 128

NC = 2
NS = 16
NW = NC * NS
CHUNK = 128
G = 16
NCHUNKS = 2560
EPAD = NCHUNKS * CHUNK
NB = NCHUNKS // G
BITERS = NB // NW
ZCH = 80
NZ = N // ZCH
ZITERS = (NZ + NS - 1) // NS

ROW_BLOCK = 1000


def _spmm_body(h_hbm, pack_hbm, z_hbm, out_hbm,
               idxb, src0, src1, dst0, dst1, rows0, rows1, acc_shared,
               sem0, sem1):
    BW = 3 * CHUNK
    srcv = (src0, src1)
    dstv = (dst0, dst1)
    rows = (rows0, rows1)
    sems = (sem0, sem1)
    cid = lax.axis_index("c")
    sid = lax.axis_index("s")
    wid = sid * NC + cid

    for i in range(ZITERS):
        c = sid + i * NS

        @pl.when(c < NZ)
        def _():
            pltpu.sync_copy(z_hbm, acc_shared.at[pl.ds(c * ZCH, ZCH)])

    plsc.subcore_barrier()

    def block_iter(i, carry):
        b = wid + i * NW
        pltpu.sync_copy(pack_hbm.at[pl.ds(b * (G * BW), G * BW)], idxb)

        def stage(g):
            k = g % 2
            goff = g * BW
            for j in range(CHUNK // 16):
                sl = pl.ds(j * 16, 16)
                srcv[k][sl] = idxb[pl.ds(goff + j * 16, 16)]
                dstv[k][sl] = idxb[pl.ds(goff + CHUNK + j * 16, 16)]
            return pltpu.async_copy(h_hbm.at[srcv[k]], rows[k], sems[k])

        gd = [None] * G
        gd[0] = stage(0)
        for g in range(G):
            k = g % 2
            if g + 1 < G:
                gd[g + 1] = stage(g + 1)
            gd[g].wait()
            rows_v = rows[k]
            goff = g * BW

            def scale(e, inner):
                wbits = plsc.load_gather(
                    idxb, [jnp.full((16,), goff + 2 * CHUNK, jnp.int32) + e])
                wvec = plsc.bitcast(wbits, jnp.float32)
                for j in range(D // 16):
                    sl = pl.ds(j * 16, 16)
                    rows_v[e, sl] = rows_v[e, sl] * wvec
                return inner

            lax.fori_loop(0, CHUNK, scale, 0, unroll=4)
            pltpu.sync_copy(rows_v, acc_shared.at[dstv[k]], add=True)
        return carry

    lax.fori_loop(0, BITERS, block_iter, 0)
    plsc.subcore_barrier()

    for i in range(ZITERS):
        c = sid + i * NS

        @pl.when(c < NZ)
        def _():
            pltpu.sync_copy(acc_shared.at[pl.ds(c * ZCH, ZCH)],
                            out_hbm.at[cid, pl.ds(c * ZCH, ZCH)])


@jax.jit
def _spmm(h, pack, zeros):
    mesh = plsc.VectorSubcoreMesh(core_axis_name="c", subcore_axis_name="s")
    f = pl.kernel(
        _spmm_body,
        out_type=jax.ShapeDtypeStruct((NC, N, D), jnp.float32),
        mesh=mesh,
        compiler_params=pltpu.CompilerParams(needs_layout_passes=False),
        scratch_types=[
            pltpu.VMEM((G * 3 * CHUNK,), jnp.int32),
            pltpu.VMEM((CHUNK,), jnp.int32),
            pltpu.VMEM((CHUNK,), jnp.int32),
            pltpu.VMEM((CHUNK,), jnp.int32),
            pltpu.VMEM((CHUNK,), jnp.int32),
            pltpu.VMEM((CHUNK, D), jnp.float32),
            pltpu.VMEM((CHUNK, D), jnp.float32),
            pltpu.VMEM_SHARED((N, D), jnp.float32),
            pltpu.SemaphoreType.DMA,
            pltpu.SemaphoreType.DMA,
        ],
    )
    return f(h, pack, zeros)


def _fc1_body(x_ref, w_ref, b_ref, o_ref):
    o_ref[...] = (
        jnp.dot(x_ref[...], w_ref[...], preferred_element_type=jnp.float32)
        + b_ref[...]
    )


@jax.jit
def _fc1(x, w, b):
    return pl.pallas_call(
        _fc1_body,
        grid=(N // ROW_BLOCK,),
        in_specs=[
            pl.BlockSpec((ROW_BLOCK, D), lambda i: (i, 0)),
            pl.BlockSpec((D, D), lambda i: (0, 0)),
            pl.BlockSpec((1, D), lambda i: (0, 0)),
        ],
        out_specs=pl.BlockSpec((ROW_BLOCK, D), lambda i: (i, 0)),
        out_shape=jax.ShapeDtypeStruct((N, D), jnp.float32),
    )(x, w, b)


def _fc2_body(p_ref, w_ref, b_ref, o_ref):
    h = jnp.maximum(p_ref[0] + p_ref[1], 0.0)
    y = jnp.dot(h, w_ref[...], preferred_element_type=jnp.float32) + b_ref[...]
    m = jnp.max(y, axis=1, keepdims=True)
    s = y - m
    o_ref[...] = s - jnp.log(jnp.sum(jnp.exp(s), axis=1, keepdims=True))


@jax.jit
def _fc2(parts, w, b):
    return pl.pallas_call(
        _fc2_body,
        grid=(N // ROW_BLOCK,),
        in_specs=[
            pl.BlockSpec((NC, ROW_BLOCK, D), lambda i: (0, i, 0)),
            pl.BlockSpec((D, D), lambda i: (0, 0)),
            pl.BlockSpec((1, D), lambda i: (0, 0)),
        ],
        out_specs=pl.BlockSpec((ROW_BLOCK, D), lambda i: (i, 0)),
        out_shape=jax.ShapeDtypeStruct((N, D), jnp.float32),
    )(parts, w, b)


def kernel(features, edge_index, edge_weight, W1, b1, W2, b2):
    pad = EPAD - E
    fill = (jnp.arange(pad, dtype=jnp.int32) * 16) % N
    src = jnp.concatenate([edge_index[0].astype(jnp.int32), fill])
    dst = jnp.concatenate([edge_index[1].astype(jnp.int32), fill])
    wbits = jax.lax.bitcast_convert_type(
        jnp.pad(edge_weight, (0, pad)), jnp.int32)
    pack = jnp.stack([src.reshape(NCHUNKS, CHUNK),
                      dst.reshape(NCHUNKS, CHUNK),
                      wbits.reshape(NCHUNKS, CHUNK)],
                     axis=1).reshape(NCHUNKS * 3 * CHUNK)
    h = _fc1(features, W1, b1.reshape(1, D))
    zeros = jnp.zeros((ZCH, D), jnp.float32)
    parts = _spmm(h, pack, zeros)
    return _fc2(parts, W2, b2.reshape(1, D))
